# baseline (speedup 1.0000x reference)
import functools

import jax
import jax.numpy as jnp
from jax import lax
from jax.experimental import pallas as pl
from jax.experimental.pallas import tpu as pltpu
from jax.experimental.pallas import tpu_sc as plsc

B, S, NC, NN, V, D = 1024, 20, 26, 13, 100000, 32
EPS = 1e-12

N = B * S * NC
NW = 32
PER_W = N // NW
CHUNK = 128
NCHUNK = PER_W // CHUNK
INNER = 13
NGROUP = NCHUNK // INNER
GROUP_ROWS = INNER * CHUNK


def _sc_gather(idx3, table):
  mesh = plsc.VectorSubcoreMesh(core_axis_name="c", subcore_axis_name="s")

  @functools.partial(
      pl.kernel,
      mesh=mesh,
      out_type=jax.ShapeDtypeStruct((N, D), jnp.float32),
      compiler_params=pltpu.CompilerParams(use_tc_tiling_on_sc=False),
      scratch_types=[
          pltpu.VMEM((NCHUNK, CHUNK), jnp.int32),
          pltpu.VMEM((GROUP_ROWS, D), jnp.float32),
          pltpu.SemaphoreType.DMA,
      ],
  )
  def k(idx_hbm, table_hbm, out_hbm, idx_v, rows_v, sem):
    wid = lax.axis_index("s") * 2 + lax.axis_index("c")
    pltpu.sync_copy(idx_hbm.at[wid], idx_v)
    row_base = wid * PER_W

    def body(g, carry):
      cps = []
      for j in range(INNER):
        cps.append(pltpu.async_copy(
            table_hbm.at[idx_v.at[g * INNER + j]],
            rows_v.at[pl.ds(j * CHUNK, CHUNK)],
            sem))
      for cp in cps:
        cp.wait()
      pltpu.sync_copy(rows_v,
                      out_hbm.at[pl.ds(row_base + g * GROUP_ROWS, GROUP_ROWS)])
      return carry

    lax.fori_loop(0, NGROUP, body, None)

  return k(idx3, table)


def _ln_body(g_ref, xn_ref, nw_ref, w_ref, b_ref, out_ref):
  g = g_ref[...]
  xn = xn_ref[...]
  nw = nw_ref[...]
  xe = xn * nw[None, :, :]
  x = jnp.concatenate([g, xe], axis=1)
  u = jnp.mean(x, axis=-1, keepdims=True)
  t = x - u
  s = jnp.mean(t * t, axis=-1, keepdims=True)
  xh = t * lax.rsqrt(s + EPS)
  out_ref[...] = w_ref[...] * xh + b_ref[...]


def _tc_ln(g3, xn3, num_weight, ln_w, ln_b):
  R = 256
  M = B * S
  grid = (M // R,)
  return pl.pallas_call(
      _ln_body,
      grid=grid,
      in_specs=[
          pl.BlockSpec((R, NC, D), lambda i: (i, 0, 0)),
          pl.BlockSpec((R, NN, 1), lambda i: (i, 0, 0)),
          pl.BlockSpec((NN, D), lambda i: (0, 0)),
          pl.BlockSpec((1, 1, D), lambda i: (0, 0, 0)),
          pl.BlockSpec((1, 1, D), lambda i: (0, 0, 0)),
      ],
      out_specs=pl.BlockSpec((R, NC + NN, D), lambda i: (i, 0, 0)),
      out_shape=jax.ShapeDtypeStruct((M, NC + NN, D), jnp.float32),
      compiler_params=pltpu.CompilerParams(
          dimension_semantics=("arbitrary",)),
  )(g3, xn3, num_weight, ln_w, ln_b)


def kernel(xc, xn, cls_tables, num_weight, ln_weight, ln_bias):
  table = cls_tables.reshape(NC * V, D)
  offs = (jnp.arange(NC, dtype=jnp.int32) * V)[None, None, :]
  idx3 = (xc + offs).reshape(NW, NCHUNK, CHUNK)
  g = _sc_gather(idx3, table)
  g3 = g.reshape(B * S, NC, D)
  xn3 = xn.reshape(B * S, NN, 1)
  out = _tc_ln(g3, xn3, num_weight,
               ln_weight.reshape(1, 1, D), ln_bias.reshape(1, 1, D))
  return out.reshape(B, S, NC + NN, D)


# speedup vs baseline: 1.3570x; 1.3570x over previous
import functools

import jax
import jax.numpy as jnp
from jax import lax
from jax.experimental import pallas as pl
from jax.experimental.pallas import tpu as pltpu
from jax.experimental.pallas import tpu_sc as plsc

B, S, NC, NN, V, D = 1024, 20, 26, 13, 100000, 32
NF = NC + NN
EPS = 1e-12

NW = 32
NPAIR = S * NC
MAXK = (NPAIR + NW - 1) // NW


def _sc_gather(xq, tt):
  mesh = plsc.VectorSubcoreMesh(core_axis_name="c", subcore_axis_name="s")

  @functools.partial(
      pl.kernel,
      mesh=mesh,
      out_type=jax.ShapeDtypeStruct((S, NC, D, 8, 128), jnp.float32),
      compiler_params=pltpu.CompilerParams(use_tc_tiling_on_sc=False),
      scratch_types=[
          pltpu.VMEM((2, 8, 128), jnp.int32),
          pltpu.VMEM((2, D, 8, 128), jnp.float32),
          pltpu.SemaphoreType.DMA,
          pltpu.SemaphoreType.DMA,
      ],
  )
  def k(xq_hbm, tt_hbm, gt_hbm, idx_v, vals_v, sem, dsem):
    wid = lax.axis_index("s") * 2 + lax.axis_index("c")

    def pair(i, q):
      p = wid + NW * i
      idx_q = idx_v.at[q]
      vals_q = vals_v.at[q]

      @pl.when(p < NPAIR)
      def _():
        s = p // NC
        c = p - s * NC

        @pl.when(i >= 2)
        def _():
          pltpu.make_async_copy(vals_q, gt_hbm.at[0, 0], dsem).wait()

        pltpu.sync_copy(xq_hbm.at[c, s], idx_q)

        def dbody(d, carry):
          for j in range(8):
            pltpu.async_copy(
                tt_hbm.at[c, d].at[idx_q.at[j]], vals_q.at[d, j], sem)
          @pl.when(d >= 2)
          def _():
            pltpu.make_async_copy(gt_hbm.at[0, 0, 0], vals_q.at[d - 2], sem
                                  ).wait()
          return carry

        lax.fori_loop(0, D, dbody, None)
        for dd in (D - 2, D - 1):
          pltpu.make_async_copy(gt_hbm.at[0, 0, 0], vals_q.at[dd], sem).wait()
        pltpu.async_copy(vals_q, gt_hbm.at[s, c], dsem)

    def body(kk, carry):
      pair(2 * kk, 0)
      pair(2 * kk + 1, 1)
      return carry

    lax.fori_loop(0, (MAXK + 1) // 2, body, None)
    pltpu.make_async_copy(vals_v.at[0], gt_hbm.at[0, 0], dsem).wait()
    pltpu.make_async_copy(vals_v.at[1], gt_hbm.at[0, 0], dsem).wait()

  return k(xq, tt)


def _ln_body(gt_ref, xn_ref, nw_ref, w_ref, b_ref, out_ref):
  g = gt_ref[0]
  xnv = xn_ref[0]
  nw = nw_ref[...]
  xe = xnv[:, None, :] * nw[:, :, None]
  x = jnp.concatenate([g, xe], axis=0)
  u = jnp.mean(x, axis=1, keepdims=True)
  t = x - u
  s = jnp.mean(t * t, axis=1, keepdims=True)
  xh = t * lax.rsqrt(s + EPS)
  out_ref[0] = w_ref[...][None, :, :] * xh + b_ref[...][None, :, :]


def _tc_ln(gt4, xnT, nw, w2, b2):
  Bb = 512
  grid = (S, B // Bb)
  return pl.pallas_call(
      _ln_body,
      grid=grid,
      in_specs=[
          pl.BlockSpec((1, NC, D, Bb), lambda i, j: (i, 0, 0, j)),
          pl.BlockSpec((1, NN, Bb), lambda i, j: (i, 0, j)),
          pl.BlockSpec((NN, D), lambda i, j: (0, 0)),
          pl.BlockSpec((D, 1), lambda i, j: (0, 0)),
          pl.BlockSpec((D, 1), lambda i, j: (0, 0)),
      ],
      out_specs=pl.BlockSpec((1, NF, D, Bb), lambda i, j: (i, 0, 0, j)),
      out_shape=jax.ShapeDtypeStruct((S, NF, D, B), jnp.float32),
      compiler_params=pltpu.CompilerParams(
          dimension_semantics=("arbitrary", "arbitrary")),
  )(gt4, xnT, nw, w2, b2)


def kernel(xc, xn, cls_tables, num_weight, ln_weight, ln_bias):
  tt = jnp.transpose(cls_tables, (0, 2, 1))
  xq = jnp.transpose(xc, (2, 1, 0)).reshape(NC, S, 8, 128)
  gt = _sc_gather(xq, tt)
  gt4 = gt.reshape(S, NC, D, B)
  xnT = jnp.transpose(xn, (1, 2, 0))
  o = _tc_ln(gt4, xnT, num_weight,
             ln_weight.reshape(D, 1), ln_bias.reshape(D, 1))
  return jnp.transpose(o, (3, 0, 1, 2))
